# baseline (speedup 1.0000x reference)
import functools

import jax
import jax.numpy as jnp
from jax import lax
from jax.experimental import pallas as pl
from jax.experimental.pallas import tpu as pltpu
from jax.experimental.pallas import tpu_sc as plsc

D = 64
MAXLEN = 200
NC = 2
NS = 16
NW = NC * NS
BT = 128


@jax.jit
def _run(x5, xpar, tok2, posr):
    B = NW * BT
    mesh = plsc.VectorSubcoreMesh(core_axis_name="c", subcore_axis_name="s")

    @functools.partial(
        pl.kernel,
        mesh=mesh,
        out_type=jax.ShapeDtypeStruct((MAXLEN, D // 8, NW, 8, BT), jnp.float32),
        scratch_types=[
            pltpu.VMEM((MAXLEN // 8, 8, BT), jnp.int32),
            pltpu.VMEM((MAXLEN * D,), jnp.float32),
            pltpu.VMEM((BT, 2 * D), jnp.float32),
            pltpu.VMEM((BT, 2 * D), jnp.float32),
            pltpu.VMEM((BT,), jnp.int32),
            pltpu.VMEM((BT,), jnp.int32),
            pltpu.VMEM((D, BT + 1), jnp.float32),
            pltpu.VMEM((D, BT + 1), jnp.float32),
            pltpu.SemaphoreType.DMA,
            pltpu.SemaphoreType.DMA,
            pltpu.SemaphoreType.DMA,
            pltpu.SemaphoreType.DMA,
        ],
        compiler_params=pltpu.CompilerParams(use_tc_tiling_on_sc=False,
                                             needs_layout_passes=False),
    )
    def k(x_hbm, xpar_hbm, tok_hbm, pos_hbm, out_hbm,
          idxv, posv, buf0, buf1, pvm0, pvm1, obuf0, obuf1,
          gsem0, gsem1, wsem0, wsem1):
        w = lax.axis_index("s") * NC + lax.axis_index("c")
        pvm = (pvm0, pvm1)
        buf = (buf0, buf1)
        obuf = (obuf0, obuf1)
        gsem = (gsem0, gsem1)
        wsem = (wsem0, wsem1)

        pltpu.sync_copy(x_hbm.at[w], idxv)
        pltpu.sync_copy(pos_hbm, posv)
        iota16 = lax.iota(jnp.int32, 16)
        zeros16 = jnp.zeros((16,), jnp.int32)
        riota = [iota16 + (16 * g) for g in range(BT // 16)]

        def idx_ref(s):
            return idxv.at[s // 8, s % 8]

        def fire_gather(p, s):
            pltpu.sync_copy(xpar_hbm.at[w, s // 8, s % 8], pvm[p])
            pltpu.async_copy(tok_hbm.at[idx_ref(s)], buf[p], gsem[p])

        def wait_gather(p, s):
            pltpu.make_async_copy(tok_hbm.at[idx_ref(s)], buf[p],
                                  gsem[p]).wait()

        def fire_write(p, s):
            for c1 in range(D // 8):
                pltpu.async_copy(obuf[p].at[pl.ds(8 * c1, 8), pl.ds(0, BT)],
                                 out_hbm.at[s, c1, w], wsem[p])

        def wait_write(p, s):
            for c1 in range(D // 8):
                pltpu.make_async_copy(
                    obuf[p].at[pl.ds(8 * c1, 8), pl.ds(0, BT)],
                    out_hbm.at[s, c1, w], wsem[p]).wait()

        def transpose_add(p, s):
            bp, op = buf[p], obuf[p]
            sD = s * D
            pv = [posv[pl.ds(sD + 16 * cg, 16)] for cg in range(D // 16)]

            pb = pvm[p]

            @plsc.parallel_loop(0, BT, unroll=2)
            def rbody(r):
                rr = zeros16 + r
                sel = plsc.load_gather(pb, [rr]) > 0
                for cg in range(D // 16):
                    lo = bp[r, pl.ds(16 * cg, 16)]
                    hi = bp[r, pl.ds(D + 16 * cg, 16)]
                    v = jnp.where(sel, hi, lo) + pv[cg]
                    plsc.store_scatter(op, [riota[cg], rr], v)

        fire_gather(0, 0)

        def body(t, carry):
            s0 = 2 * t
            fire_gather(1, s0 + 1)
            wait_gather(0, s0)

            @pl.when(t >= 1)
            def _():
                wait_write(0, s0 - 2)

            transpose_add(0, s0)
            fire_write(0, s0)

            @pl.when(t < MAXLEN // 2 - 1)
            def _():
                fire_gather(0, s0 + 2)

            wait_gather(1, s0 + 1)

            @pl.when(t >= 1)
            def _():
                wait_write(1, s0 - 1)

            transpose_add(1, s0 + 1)
            fire_write(1, s0 + 1)
            return carry

        lax.fori_loop(0, MAXLEN // 2, body, 0)
        wait_write(0, MAXLEN - 2)
        wait_write(1, MAXLEN - 1)

    return k(x5, xpar, tok2, posr)


def kernel(x, token_table, pos_table):
    B, S = x.shape
    xt = x.T.reshape(S // 8, 8, B // BT, BT).transpose(2, 0, 1, 3)
    x5 = xt >> 1
    xpar = xt & 1
    tok2 = token_table.reshape(-1, 2 * D)
    posr = pos_table.reshape(-1)
    out5 = _run(x5, xpar, tok2, posr)
    return out5.transpose(2, 4, 0, 1, 3).reshape(B, S, D)


# speedup vs baseline: 1.1619x; 1.1619x over previous
import functools

import jax
import jax.numpy as jnp
from jax import lax
from jax.experimental import pallas as pl
from jax.experimental.pallas import tpu as pltpu
from jax.experimental.pallas import tpu_sc as plsc

D = 64
MAXLEN = 200
NC = 2
NS = 16
NW = NC * NS
BT = 128


@jax.jit
def _run(x5, token_table, posr):
    B = NW * BT
    mesh = plsc.VectorSubcoreMesh(core_axis_name="c", subcore_axis_name="s")

    @functools.partial(
        pl.kernel,
        mesh=mesh,
        out_type=jax.ShapeDtypeStruct((MAXLEN, D // 8, NW, 8, BT), jnp.float32),
        scratch_types=[
            pltpu.VMEM((MAXLEN // 8, 8, BT), jnp.int32),
            pltpu.VMEM((MAXLEN * D,), jnp.float32),
            pltpu.VMEM((BT, D), jnp.float32),
            pltpu.VMEM((BT, D), jnp.float32),
            pltpu.VMEM((D, BT + 1), jnp.float32),
            pltpu.VMEM((D, BT + 1), jnp.float32),
            pltpu.SemaphoreType.DMA,
            pltpu.SemaphoreType.DMA,
            pltpu.SemaphoreType.DMA,
            pltpu.SemaphoreType.DMA,
        ],
        compiler_params=pltpu.CompilerParams(use_tc_tiling_on_sc=False,
                                             needs_layout_passes=False),
    )
    def k(x_hbm, tok_hbm, pos_hbm, out_hbm,
          idxv, posv, buf0, buf1, obuf0, obuf1, gsem0, gsem1, wsem0, wsem1):
        w = lax.axis_index("s") * NC + lax.axis_index("c")
        buf = (buf0, buf1)
        obuf = (obuf0, obuf1)
        gsem = (gsem0, gsem1)
        wsem = (wsem0, wsem1)

        pltpu.sync_copy(x_hbm.at[w], idxv)
        pltpu.sync_copy(pos_hbm, posv)
        iota16 = lax.iota(jnp.int32, 16)
        zeros16 = jnp.zeros((16,), jnp.int32)
        riota = [iota16 + (16 * g) for g in range(BT // 16)]

        def idx_ref(s):
            return idxv.at[s // 8, s % 8]

        def fire_gather(p, s):
            pltpu.async_copy(tok_hbm.at[idx_ref(s)], buf[p], gsem[p])

        def wait_gather(p, s):
            pltpu.make_async_copy(tok_hbm.at[idx_ref(s)], buf[p],
                                  gsem[p]).wait()

        def fire_write(p, s):
            for c1 in range(D // 8):
                pltpu.async_copy(obuf[p].at[pl.ds(8 * c1, 8), pl.ds(0, BT)],
                                 out_hbm.at[s, c1, w], wsem[p])

        def wait_write(p, s):
            for c1 in range(D // 8):
                pltpu.make_async_copy(
                    obuf[p].at[pl.ds(8 * c1, 8), pl.ds(0, BT)],
                    out_hbm.at[s, c1, w], wsem[p]).wait()

        def transpose_add(p, s):
            bp, op = buf[p], obuf[p]
            sD = s * D
            pv = [posv[pl.ds(sD + 16 * cg, 16)] for cg in range(D // 16)]

            @plsc.parallel_loop(0, BT, unroll=2)
            def rbody(r):
                rr = zeros16 + r
                for cg in range(D // 16):
                    v = bp[r, pl.ds(16 * cg, 16)] + pv[cg]
                    plsc.store_scatter(op, [riota[cg], rr], v)

        fire_gather(0, 0)

        def body(t, carry):
            s0 = 2 * t
            fire_gather(1, s0 + 1)
            wait_gather(0, s0)

            @pl.when(t >= 1)
            def _():
                wait_write(0, s0 - 2)

            transpose_add(0, s0)
            fire_write(0, s0)

            @pl.when(t < MAXLEN // 2 - 1)
            def _():
                fire_gather(0, s0 + 2)

            wait_gather(1, s0 + 1)

            @pl.when(t >= 1)
            def _():
                wait_write(1, s0 - 1)

            transpose_add(1, s0 + 1)
            fire_write(1, s0 + 1)
            return carry

        lax.fori_loop(0, MAXLEN // 2, body, 0)
        wait_write(0, MAXLEN - 2)
        wait_write(1, MAXLEN - 1)

    return k(x5, token_table, posr)


def kernel(x, token_table, pos_table):
    B, S = x.shape
    x5 = x.T.reshape(S // 8, 8, B // BT, BT).transpose(2, 0, 1, 3)
    posr = pos_table.reshape(-1)
    out5 = _run(x5, token_table, posr)
    return out5.transpose(2, 4, 0, 1, 3).reshape(B, S, D)
